# Initial kernel scaffold; baseline (speedup 1.0000x reference)
import jax
import jax.numpy as jnp
from jax.experimental import pallas as pl


def kernel(input, W):
    raise NotImplementedError("write your pallas kernel here")


# speedup vs baseline: 5.1941x; 5.1941x over previous
import functools

import jax
import jax.numpy as jnp
from jax import lax
from jax.experimental import pallas as pl
from jax.experimental.pallas import tpu as pltpu
from jax.experimental.pallas import tpu_sc as plsc

NC = 2
NS = 16
L = 16
NW = NC * NS

N = 16384 * 200
PER_W = N // NW
CHUNK = 2048
N_CHUNKS = PER_W // CHUNK
VECS = CHUNK // L


def _body(idx_hbm, w_hbm, out_hbm, w_v, idx_v, out_v):
    wid = lax.axis_index("s") * NC + lax.axis_index("c")
    base = wid * PER_W

    pltpu.sync_copy(w_hbm, w_v)

    lane = lax.iota(jnp.int32, L)
    half = lane >> 3
    col = lane & 7

    def chunk_body(c, _):
        start = base + c * CHUNK
        pltpu.sync_copy(idx_hbm.at[pl.ds(start, CHUNK)], idx_v)

        def vec_body(v, _):
            for r in range(8):
                rows = plsc.load_gather(idx_v, [half + (v * L + 2 * r)])
                vals = plsc.load_gather(w_v, [rows, col])
                out_v[pl.ds(v * 8 * L + r * L, L)] = vals
            return 0

        lax.fori_loop(0, VECS, vec_body, 0)
        pltpu.sync_copy(out_v, out_hbm.at[pl.ds(start * 8, CHUNK * 8)])
        return 0

    lax.fori_loop(0, N_CHUNKS, chunk_body, 0)


@jax.jit
def _run(idx_flat, W):
    mesh = plsc.VectorSubcoreMesh(core_axis_name="c", subcore_axis_name="s",
                                  num_cores=NC, num_subcores=NS)
    f = pl.kernel(
        _body,
        out_type=jax.ShapeDtypeStruct((N * 8,), jnp.float32),
        mesh=mesh,
        scratch_types=[
            pltpu.VMEM((32, 8), jnp.float32),
            pltpu.VMEM((CHUNK,), jnp.int32),
            pltpu.VMEM((CHUNK * 8,), jnp.float32),
        ],
        compiler_params=pltpu.CompilerParams(needs_layout_passes=False),
    )
    return f(idx_flat, W)


def kernel(input, W):
    out_flat = _run(input.reshape(-1), W)
    return out_flat.reshape(16384, 200, 8)


# speedup vs baseline: 5.3052x; 1.0214x over previous
import jax
import jax.numpy as jnp
from jax import lax
from jax.experimental import pallas as pl
from jax.experimental.pallas import tpu as pltpu
from jax.experimental.pallas import tpu_sc as plsc

NC = 2
NS = 16
L = 16
NW = NC * NS

B, T, D = 16384, 200, 8
N = B * T
PER_W = N // NW
CHUNK = 2048
N_CHUNKS = PER_W // CHUNK
assert N_CHUNKS % 2 == 0 and PER_W % CHUNK == 0
VECS = CHUNK // L


def _body(idx_flat, w_hbm, out_flat, w_v, idx_v0, idx_v1, out_v0, out_v1,
          sem0, sem1):
    wid = lax.axis_index("s") * NC + lax.axis_index("c")
    base = wid * PER_W

    pltpu.sync_copy(w_hbm, w_v)

    lane = lax.iota(jnp.int32, L)
    half = lane >> 3
    col = lane & 7
    sems = (sem0, sem1)
    idx_bufs = (idx_v0, idx_v1)
    out_bufs = (out_v0, out_v1)

    def outer(c0):
        for b in range(2):
            c = c0 + b
            start = base + c * CHUNK
            pltpu.sync_copy(idx_flat.at[pl.ds(start, CHUNK)], idx_bufs[b])

            @pl.when(c >= 2)
            def _():
                pltpu.make_async_copy(
                    out_bufs[b],
                    out_flat.at[pl.ds((start - 2 * CHUNK) * D, CHUNK * D)],
                    sems[b],
                ).wait()

            def vec_body(v, _):
                for r in range(8):
                    rows = plsc.load_gather(idx_bufs[b],
                                            [half + (v * L + 2 * r)])
                    vals = plsc.load_gather(w_v, [rows, col])
                    out_bufs[b][pl.ds(v * 8 * L + r * L, L)] = vals
                return 0

            lax.fori_loop(0, VECS, vec_body, 0, unroll=2)
            pltpu.async_copy(out_bufs[b],
                             out_flat.at[pl.ds(start * D, CHUNK * D)],
                             sems[b])

    pl.loop(0, N_CHUNKS, step=2)(outer)

    for b in range(2):
        c = N_CHUNKS - 2 + b
        start = base + c * CHUNK
        pltpu.make_async_copy(
            out_bufs[b],
            out_flat.at[pl.ds(start * D, CHUNK * D)],
            sems[b],
        ).wait()


@jax.jit
def _run(idx, W):
    mesh = plsc.VectorSubcoreMesh(core_axis_name="c", subcore_axis_name="s",
                                  num_cores=NC, num_subcores=NS)
    f = pl.kernel(
        _body,
        out_type=jax.ShapeDtypeStruct((N * D,), jnp.float32),
        mesh=mesh,
        scratch_types=[
            pltpu.VMEM((32, 8), jnp.float32),
            pltpu.VMEM((CHUNK,), jnp.int32),
            pltpu.VMEM((CHUNK,), jnp.int32),
            pltpu.VMEM((CHUNK * D,), jnp.float32),
            pltpu.VMEM((CHUNK * D,), jnp.float32),
            pltpu.SemaphoreType.DMA,
            pltpu.SemaphoreType.DMA,
        ],
        compiler_params=pltpu.CompilerParams(needs_layout_passes=False),
    )
    return f(idx, W)


def kernel(input, W):
    return _run(input.reshape(-1), W).reshape(B, T, D)


# speedup vs baseline: 26.9991x; 5.0891x over previous
import jax
import jax.numpy as jnp
from jax import lax
from jax.experimental import pallas as pl
from jax.experimental.pallas import tpu as pltpu
from jax.experimental.pallas import tpu_sc as plsc

NC = 2
NS = 16
L = 16
NW = NC * NS

B, T, D = 16384, 200, 8
BT = B // 128
BT_W = BT // NW
BPW = BT_W * 128
T_C = 8
N_CHUNKS = T // T_C
N_MAIN = N_CHUNKS - 1
K = 128 // L


def _body(idx_hbm, w_hbm, out_hbm, w_v, idx_v0, idx_v1, out_v0, out_v1,
          sem0, sem1):
    wid = lax.axis_index("s") * NC + lax.axis_index("c")
    row0 = wid * BPW

    pltpu.sync_copy(w_hbm, w_v)

    lane = lax.iota(jnp.int32, L)
    sems = (sem0, sem1)
    idx_bufs = (idx_v0, idx_v1)
    out_bufs = (out_v0, out_v1)

    def outer(c0):
        for bb in range(2):
            c = c0 + bb
            t0 = c * T_C
            pltpu.sync_copy(
                idx_hbm.at[pl.ds(row0, BPW), pl.ds(t0, T_C)], idx_bufs[bb])

            @pl.when(c >= 2)
            def _():
                pltpu.make_async_copy(
                    out_bufs[bb],
                    out_hbm.at[pl.ds(t0 - 2 * T_C, T_C),
                               pl.ds(wid * BT_W, BT_W)],
                    sems[bb],
                ).wait()

            def t_body(tt, _):
                tcol = jnp.full((L,), tt, jnp.int32)

                def k_body(k, _):
                    for btl in range(BT_W):
                        rvec = lane + (btl * 128 + k * L)
                        rows = plsc.load_gather(idx_bufs[bb], [rvec, tcol])
                        for j in range(D):
                            vals = plsc.load_gather(
                                w_v, [rows, jnp.full((L,), j, jnp.int32)])
                            out_bufs[bb][tt, btl, j, pl.ds(k * L, L)] = vals
                    return 0

                lax.fori_loop(0, K, k_body, 0)
                return 0

            lax.fori_loop(0, T_C, t_body, 0)
            pltpu.async_copy(
                out_bufs[bb],
                out_hbm.at[pl.ds(t0, T_C), pl.ds(wid * BT_W, BT_W)],
                sems[bb])

    pl.loop(0, N_MAIN, step=2)(outer)

    t_tail = N_MAIN * T_C
    pltpu.make_async_copy(
        out_bufs[0],
        out_hbm.at[pl.ds(t_tail - 2 * T_C, T_C), pl.ds(wid * BT_W, BT_W)],
        sems[0],
    ).wait()
    pltpu.sync_copy(idx_hbm.at[pl.ds(row0, BPW), pl.ds(t_tail, T_C)],
                    idx_bufs[0])

    def t_body_tail(tt, _):
        tcol = jnp.full((L,), tt, jnp.int32)

        def k_body(k, _):
            for btl in range(BT_W):
                rvec = lane + (btl * 128 + k * L)
                rows = plsc.load_gather(idx_bufs[0], [rvec, tcol])
                for j in range(D):
                    vals = plsc.load_gather(
                        w_v, [rows, jnp.full((L,), j, jnp.int32)])
                    out_bufs[0][tt, btl, j, pl.ds(k * L, L)] = vals
            return 0

        lax.fori_loop(0, K, k_body, 0)
        return 0

    lax.fori_loop(0, T_C, t_body_tail, 0)
    pltpu.async_copy(
        out_bufs[0],
        out_hbm.at[pl.ds(t_tail, T_C), pl.ds(wid * BT_W, BT_W)],
        sems[0])

    pltpu.make_async_copy(
        out_bufs[1],
        out_hbm.at[pl.ds(t_tail - T_C, T_C), pl.ds(wid * BT_W, BT_W)],
        sems[1],
    ).wait()
    pltpu.make_async_copy(
        out_bufs[0],
        out_hbm.at[pl.ds(t_tail, T_C), pl.ds(wid * BT_W, BT_W)],
        sems[0],
    ).wait()


@jax.jit
def _run(idx, W):
    mesh = plsc.VectorSubcoreMesh(core_axis_name="c", subcore_axis_name="s",
                                  num_cores=NC, num_subcores=NS)
    f = pl.kernel(
        _body,
        out_type=jax.ShapeDtypeStruct((T, BT, D, 128), jnp.float32),
        mesh=mesh,
        scratch_types=[
            pltpu.VMEM((32, 8), jnp.float32),
            pltpu.VMEM((BPW, T_C), jnp.int32),
            pltpu.VMEM((BPW, T_C), jnp.int32),
            pltpu.VMEM((T_C, BT_W, D, 128), jnp.float32),
            pltpu.VMEM((T_C, BT_W, D, 128), jnp.float32),
            pltpu.SemaphoreType.DMA,
            pltpu.SemaphoreType.DMA,
        ],
        compiler_params=pltpu.CompilerParams(needs_layout_passes=False,
                                            use_tc_tiling_on_sc=False),
    )
    return f(idx, W)


def kernel(input, W):
    x = _run(input, W)
    return jnp.transpose(x, (1, 3, 0, 2)).reshape(B, T, D)


# speedup vs baseline: 59.1239x; 2.1898x over previous
import jax
import jax.numpy as jnp
from jax import lax
from jax.experimental import pallas as pl
from jax.experimental.pallas import tpu as pltpu
from jax.experimental.pallas import tpu_sc as plsc

NC = 2
NS = 16
L = 16
NW = NC * NS

B, T, D = 16384, 200, 8
BT = B // 128
BT_W = BT // NW
BPW = BT_W * 128
T_C = 8
N_CHUNKS = T // T_C
N_MAIN = N_CHUNKS - 1
K = 128 // L


def _compute_chunk(idx_buf, out_buf, w_v, lane):
    jcols = [jnp.full((L,), j, jnp.int32) for j in range(D)]

    def t_body(tt, _):
        tcol = jnp.full((L,), tt, jnp.int32)

        def k_body(k, _):
            rows = [plsc.load_gather(idx_buf, [lane + (btl * 128 + k * L),
                                               tcol])
                    for btl in range(BT_W)]
            vals = [[plsc.load_gather(w_v, [rows[btl], jcols[j]])
                     for j in range(D)] for btl in range(BT_W)]
            for btl in range(BT_W):
                for j in range(D):
                    out_buf[tt, btl, j, pl.ds(k * L, L)] = vals[btl][j]
            return 0

        lax.fori_loop(0, K, k_body, 0)
        return 0

    lax.fori_loop(0, T_C, t_body, 0)


def _body(idx_hbm, w_hbm, out_hbm, w_v, idx_v0, idx_v1, out_v0, out_v1,
          sem0, sem1):
    wid = lax.axis_index("s") * NC + lax.axis_index("c")
    row0 = wid * BPW

    pltpu.sync_copy(w_hbm, w_v)

    lane = lax.iota(jnp.int32, L)
    sems = (sem0, sem1)
    idx_bufs = (idx_v0, idx_v1)
    out_bufs = (out_v0, out_v1)

    def outer(c0):
        for bb in range(2):
            c = c0 + bb
            t0 = c * T_C
            pltpu.sync_copy(
                idx_hbm.at[pl.ds(row0, BPW), pl.ds(t0, T_C)], idx_bufs[bb])

            @pl.when(c >= 2)
            def _():
                pltpu.make_async_copy(
                    out_bufs[bb],
                    out_hbm.at[pl.ds(t0 - 2 * T_C, T_C),
                               pl.ds(wid * BT_W, BT_W)],
                    sems[bb],
                ).wait()

            _compute_chunk(idx_bufs[bb], out_bufs[bb], w_v, lane)
            pltpu.async_copy(
                out_bufs[bb],
                out_hbm.at[pl.ds(t0, T_C), pl.ds(wid * BT_W, BT_W)],
                sems[bb])

    pl.loop(0, N_MAIN, step=2)(outer)

    t_tail = N_MAIN * T_C
    pltpu.make_async_copy(
        out_bufs[0],
        out_hbm.at[pl.ds(t_tail - 2 * T_C, T_C), pl.ds(wid * BT_W, BT_W)],
        sems[0],
    ).wait()
    pltpu.sync_copy(idx_hbm.at[pl.ds(row0, BPW), pl.ds(t_tail, T_C)],
                    idx_bufs[0])

    _compute_chunk(idx_bufs[0], out_bufs[0], w_v, lane)
    pltpu.async_copy(
        out_bufs[0],
        out_hbm.at[pl.ds(t_tail, T_C), pl.ds(wid * BT_W, BT_W)],
        sems[0])

    pltpu.make_async_copy(
        out_bufs[1],
        out_hbm.at[pl.ds(t_tail - T_C, T_C), pl.ds(wid * BT_W, BT_W)],
        sems[1],
    ).wait()
    pltpu.make_async_copy(
        out_bufs[0],
        out_hbm.at[pl.ds(t_tail, T_C), pl.ds(wid * BT_W, BT_W)],
        sems[0],
    ).wait()


@jax.jit
def _run(idx, W):
    mesh = plsc.VectorSubcoreMesh(core_axis_name="c", subcore_axis_name="s",
                                  num_cores=NC, num_subcores=NS)
    f = pl.kernel(
        _body,
        out_type=jax.ShapeDtypeStruct((T, BT, D, 128), jnp.float32),
        mesh=mesh,
        scratch_types=[
            pltpu.VMEM((32, 8), jnp.float32),
            pltpu.VMEM((BPW, T_C), jnp.int32),
            pltpu.VMEM((BPW, T_C), jnp.int32),
            pltpu.VMEM((T_C, BT_W, D, 128), jnp.float32),
            pltpu.VMEM((T_C, BT_W, D, 128), jnp.float32),
            pltpu.SemaphoreType.DMA,
            pltpu.SemaphoreType.DMA,
        ],
        compiler_params=pltpu.CompilerParams(needs_layout_passes=False,
                                            use_tc_tiling_on_sc=False),
    )
    return f(idx, W)


def kernel(input, W):
    x = _run(input, W)
    return jnp.transpose(x, (1, 3, 0, 2)).reshape(B, T, D)


# speedup vs baseline: 59.8227x; 1.0118x over previous
import jax
import jax.numpy as jnp
from jax import lax
from jax.experimental import pallas as pl
from jax.experimental.pallas import tpu as pltpu
from jax.experimental.pallas import tpu_sc as plsc

NC = 2
NS = 16
L = 16
NW = NC * NS

B, T, D = 16384, 200, 8
BT = B // 128
BT_W = BT // NW
BPW = BT_W * 128
T_C = 8
N_CHUNKS = T // T_C
N_MAIN = N_CHUNKS - 1
K = 128 // L


def _compute_chunk(idx_buf, out_buf, w_v, lane):
    jcols = [jnp.full((L,), j, jnp.int32) for j in range(D)]

    def t_body(tt, _):
        tcol = jnp.full((L,), tt, jnp.int32)
        for k in range(K):
            rows = [plsc.load_gather(idx_buf, [lane + (btl * 128 + k * L),
                                               tcol])
                    for btl in range(BT_W)]
            vals = [[plsc.load_gather(w_v, [rows[btl], jcols[j]])
                     for j in range(D)] for btl in range(BT_W)]
            for btl in range(BT_W):
                for j in range(D):
                    out_buf[tt, btl, j, pl.ds(k * L, L)] = vals[btl][j]
        return 0

    lax.fori_loop(0, T_C, t_body, 0)


def _body(idx_hbm, w_hbm, out_hbm, w_v, idx_v0, idx_v1, out_v0, out_v1,
          sem0, sem1):
    wid = lax.axis_index("s") * NC + lax.axis_index("c")
    row0 = wid * BPW

    pltpu.sync_copy(w_hbm, w_v)

    lane = lax.iota(jnp.int32, L)
    sems = (sem0, sem1)
    idx_bufs = (idx_v0, idx_v1)
    out_bufs = (out_v0, out_v1)

    def outer(c0):
        for bb in range(2):
            c = c0 + bb
            t0 = c * T_C
            pltpu.sync_copy(
                idx_hbm.at[pl.ds(row0, BPW), pl.ds(t0, T_C)], idx_bufs[bb])

            @pl.when(c >= 2)
            def _():
                pltpu.make_async_copy(
                    out_bufs[bb],
                    out_hbm.at[pl.ds(t0 - 2 * T_C, T_C),
                               pl.ds(wid * BT_W, BT_W)],
                    sems[bb],
                ).wait()

            _compute_chunk(idx_bufs[bb], out_bufs[bb], w_v, lane)
            pltpu.async_copy(
                out_bufs[bb],
                out_hbm.at[pl.ds(t0, T_C), pl.ds(wid * BT_W, BT_W)],
                sems[bb])

    pl.loop(0, N_MAIN, step=2)(outer)

    t_tail = N_MAIN * T_C
    pltpu.make_async_copy(
        out_bufs[0],
        out_hbm.at[pl.ds(t_tail - 2 * T_C, T_C), pl.ds(wid * BT_W, BT_W)],
        sems[0],
    ).wait()
    pltpu.sync_copy(idx_hbm.at[pl.ds(row0, BPW), pl.ds(t_tail, T_C)],
                    idx_bufs[0])

    _compute_chunk(idx_bufs[0], out_bufs[0], w_v, lane)
    pltpu.async_copy(
        out_bufs[0],
        out_hbm.at[pl.ds(t_tail, T_C), pl.ds(wid * BT_W, BT_W)],
        sems[0])

    pltpu.make_async_copy(
        out_bufs[1],
        out_hbm.at[pl.ds(t_tail - T_C, T_C), pl.ds(wid * BT_W, BT_W)],
        sems[1],
    ).wait()
    pltpu.make_async_copy(
        out_bufs[0],
        out_hbm.at[pl.ds(t_tail, T_C), pl.ds(wid * BT_W, BT_W)],
        sems[0],
    ).wait()


@jax.jit
def _run(idx, W):
    mesh = plsc.VectorSubcoreMesh(core_axis_name="c", subcore_axis_name="s",
                                  num_cores=NC, num_subcores=NS)
    f = pl.kernel(
        _body,
        out_type=jax.ShapeDtypeStruct((T, BT, D, 128), jnp.float32),
        mesh=mesh,
        scratch_types=[
            pltpu.VMEM((32, 8), jnp.float32),
            pltpu.VMEM((BPW, T_C), jnp.int32),
            pltpu.VMEM((BPW, T_C), jnp.int32),
            pltpu.VMEM((T_C, BT_W, D, 128), jnp.float32),
            pltpu.VMEM((T_C, BT_W, D, 128), jnp.float32),
            pltpu.SemaphoreType.DMA,
            pltpu.SemaphoreType.DMA,
        ],
        compiler_params=pltpu.CompilerParams(needs_layout_passes=False,
                                            use_tc_tiling_on_sc=False),
    )
    return f(idx, W)


def kernel(input, W):
    x = _run(input, W)
    return jnp.transpose(x, (1, 3, 0, 2)).reshape(B, T, D)


# speedup vs baseline: 78.3166x; 1.3091x over previous
import jax
import jax.numpy as jnp
from jax import lax
from jax.experimental import pallas as pl
from jax.experimental.pallas import tpu as pltpu
from jax.experimental.pallas import tpu_sc as plsc

NC = 2
NS = 16
L = 16
NW = NC * NS

B, T, D = 16384, 200, 8
BT = B // 128
BT_W = BT // NW
BPW = BT_W * 128
T_C = 8
N_CHUNKS = T // T_C
N_MAIN = N_CHUNKS - 1
K = 128 // L


def _compute_chunk(idx_buf, out_buf, w_v, lane):
    jcols = [jnp.full((L,), j, jnp.int32) for j in range(D)]

    def t_body(tt, _):
        tcol = jnp.full((L,), tt, jnp.int32)
        for k in range(K):
            rows = [plsc.load_gather(idx_buf, [lane + (btl * 128 + k * L),
                                               tcol])
                    for btl in range(BT_W)]
            vals = [[plsc.load_gather(w_v, [rows[btl], jcols[j]])
                     for j in range(D)] for btl in range(BT_W)]
            for btl in range(BT_W):
                for j in range(D):
                    out_buf[tt, btl, j, pl.ds(k * L, L)] = vals[btl][j]
        return 0

    lax.fori_loop(0, T_C, t_body, 0)


def _body(idx_hbm, w_hbm, out_hbm, w_v, idx_v0, idx_v1, out_v0, out_v1,
          sem0, sem1, semi0, semi1):
    wid = lax.axis_index("s") * NC + lax.axis_index("c")
    row0 = wid * BPW

    pltpu.sync_copy(w_hbm, w_v)

    lane = lax.iota(jnp.int32, L)
    sems = (sem0, sem1)
    sems_i = (semi0, semi1)
    idx_bufs = (idx_v0, idx_v1)
    out_bufs = (out_v0, out_v1)

    pltpu.async_copy(idx_hbm.at[pl.ds(row0, BPW), pl.ds(0, T_C)],
                     idx_bufs[0], sems_i[0])

    def outer(c0):
        for bb in range(2):
            c = c0 + bb
            t0 = c * T_C
            pltpu.make_async_copy(
                idx_hbm.at[pl.ds(row0, BPW), pl.ds(t0, T_C)],
                idx_bufs[bb], sems_i[bb]).wait()

            @pl.when(c + 1 < N_CHUNKS)
            def _():
                pltpu.async_copy(
                    idx_hbm.at[pl.ds(row0 + 0, BPW),
                               pl.ds(t0 + T_C, T_C)],
                    idx_bufs[1 - bb], sems_i[1 - bb])

            @pl.when(c >= 2)
            def _():
                pltpu.make_async_copy(
                    out_bufs[bb],
                    out_hbm.at[pl.ds(t0 - 2 * T_C, T_C),
                               pl.ds(wid * BT_W, BT_W)],
                    sems[bb],
                ).wait()

            _compute_chunk(idx_bufs[bb], out_bufs[bb], w_v, lane)
            pltpu.async_copy(
                out_bufs[bb],
                out_hbm.at[pl.ds(t0, T_C), pl.ds(wid * BT_W, BT_W)],
                sems[bb])

    pl.loop(0, N_MAIN, step=2)(outer)

    t_tail = N_MAIN * T_C
    pltpu.make_async_copy(
        out_bufs[0],
        out_hbm.at[pl.ds(t_tail - 2 * T_C, T_C), pl.ds(wid * BT_W, BT_W)],
        sems[0],
    ).wait()
    pltpu.make_async_copy(
        idx_hbm.at[pl.ds(row0, BPW), pl.ds(t_tail, T_C)],
        idx_bufs[0], sems_i[0]).wait()

    _compute_chunk(idx_bufs[0], out_bufs[0], w_v, lane)
    pltpu.async_copy(
        out_bufs[0],
        out_hbm.at[pl.ds(t_tail, T_C), pl.ds(wid * BT_W, BT_W)],
        sems[0])

    pltpu.make_async_copy(
        out_bufs[1],
        out_hbm.at[pl.ds(t_tail - T_C, T_C), pl.ds(wid * BT_W, BT_W)],
        sems[1],
    ).wait()
    pltpu.make_async_copy(
        out_bufs[0],
        out_hbm.at[pl.ds(t_tail, T_C), pl.ds(wid * BT_W, BT_W)],
        sems[0],
    ).wait()


@jax.jit
def _run(idx, W):
    mesh = plsc.VectorSubcoreMesh(core_axis_name="c", subcore_axis_name="s",
                                  num_cores=NC, num_subcores=NS)
    f = pl.kernel(
        _body,
        out_type=jax.ShapeDtypeStruct((T, BT, D, 128), jnp.float32),
        mesh=mesh,
        scratch_types=[
            pltpu.VMEM((32, 8), jnp.float32),
            pltpu.VMEM((BPW, T_C), jnp.int32),
            pltpu.VMEM((BPW, T_C), jnp.int32),
            pltpu.VMEM((T_C, BT_W, D, 128), jnp.float32),
            pltpu.VMEM((T_C, BT_W, D, 128), jnp.float32),
            pltpu.SemaphoreType.DMA,
            pltpu.SemaphoreType.DMA,
            pltpu.SemaphoreType.DMA,
            pltpu.SemaphoreType.DMA,
        ],
        compiler_params=pltpu.CompilerParams(needs_layout_passes=False,
                                            use_tc_tiling_on_sc=False),
    )
    return f(idx, W)


def kernel(input, W):
    x = _run(input, W)
    return jnp.transpose(x, (1, 3, 0, 2)).reshape(B, T, D)
